# Initial kernel scaffold; baseline (speedup 1.0000x reference)
import jax
import jax.numpy as jnp
from jax.experimental import pallas as pl


def kernel(edge_index, edge_type, basis1, comp1, root1, bias1, basis2, comp2, root2, bias2):
    raise NotImplementedError("write your pallas kernel here")


# speedup vs baseline: 16.1652x; 16.1652x over previous
import functools

import jax
import jax.numpy as jnp
from jax import lax
from jax.experimental import pallas as pl
from jax.experimental.pallas import tpu as pltpu, tpu_sc as plsc

N = 50000
E = 1600000
R = 8
NB = 30
H = 16
C = 16

NC = 2
NS = 16
L = 16
NW = NC * NS

EROWS = E // 128
CHR = 4
NCHUNK = EROWS // CHR
CH_BASE = NCHUNK // NW
CH_EXTRA = NCHUNK - CH_BASE * NW

ZROW = 3128
CROW_T = (N * R) // NS

_MESH = plsc.VectorSubcoreMesh(core_axis_name="c", subcore_axis_name="s")
_SC_PARAMS = pltpu.CompilerParams(needs_layout_passes=False, use_tc_tiling_on_sc=False)


def _wid():
    return lax.axis_index("s") * NC + lax.axis_index("c")


def _chunk_range(w):
    nch = CH_BASE + (w < CH_EXTRA).astype(jnp.int32)
    start = w * CH_BASE + jnp.minimum(w, CH_EXTRA)
    return start, nch


def _cnt_body(dst2, typ2, zc, out, dstb, typb, kidx, onesb, czbuf, cacc):
    c = lax.axis_index("c")
    s = lax.axis_index("s")
    w = _wid()
    pltpu.sync_copy(zc, czbuf)
    cbase = s * CROW_T
    for k in range(24):
        pltpu.sync_copy(czbuf, cacc.at[pl.ds(cbase + k * 1024, 1024)])
    pltpu.sync_copy(czbuf.at[pl.ds(0, CROW_T - 24 * 1024)],
                    cacc.at[pl.ds(cbase + 24 * 1024, CROW_T - 24 * 1024)])
    for j in range(CHR):
        for g in range(128 // L):
            onesb[j, pl.ds(g * L, L)] = jnp.full((L,), 1.0, jnp.float32)
    plsc.subcore_barrier()

    start, nch = _chunk_range(w)

    def chunk(ci, carry):
        pltpu.sync_copy(dst2.at[start + ci], dstb)
        pltpu.sync_copy(typ2.at[start + ci], typb)
        for j in range(CHR):
            for g in range(128 // L):
                sl = pl.ds(g * L, L)
                kidx[j, sl] = dstb[j, sl] * R + typb[j, sl]
        for j in range(CHR):
            pltpu.sync_copy(onesb.at[j], cacc.at[kidx.at[j]], add=True)
        return carry

    lax.fori_loop(0, nch, chunk, 0)
    plsc.subcore_barrier()
    obase = c * (N * R) + s * CROW_T
    for k in range(24):
        pltpu.sync_copy(cacc.at[pl.ds(cbase + k * 1024, 1024)], czbuf)
        pltpu.sync_copy(czbuf, out.at[pl.ds(obase + k * 1024, 1024)])
    tail = CROW_T - 24 * 1024
    pltpu.sync_copy(cacc.at[pl.ds(cbase + 24 * 1024, tail)],
                    czbuf.at[pl.ds(0, tail)])
    pltpu.sync_copy(czbuf.at[pl.ds(0, tail)],
                    out.at[pl.ds(obase + 24 * 1024, tail)])


_cnt_kernel = functools.partial(
    pl.kernel,
    out_type=jax.ShapeDtypeStruct((NC * N * R,), jnp.float32),
    mesh=_MESH,
    compiler_params=_SC_PARAMS,
    scratch_types=[
        pltpu.VMEM((CHR, 128), jnp.int32),
        pltpu.VMEM((CHR, 128), jnp.int32),
        pltpu.VMEM((CHR, 128), jnp.int32),
        pltpu.VMEM((CHR, 128), jnp.float32),
        pltpu.VMEM((1024,), jnp.float32),
        pltpu.VMEM_SHARED((N * R,), jnp.float32),
    ],
)(_cnt_body)


def _gss_body(am, bm, src2, dst2, typ2, table, inv2, zn, out,
              srcb, dstb, typb, gidx, rows, normb, zbuf, acc):
    c = lax.axis_index("c")
    s = lax.axis_index("s")
    w = _wid()
    pltpu.sync_copy(zn, zbuf)
    zoff = jnp.minimum(s * ZROW, N - ZROW)
    for k in range(3):
        pltpu.sync_copy(zbuf, acc.at[pl.ds(zoff + k * 1024, 1024)])
    pltpu.sync_copy(zbuf.at[pl.ds(0, ZROW - 3 * 1024)],
                    acc.at[pl.ds(zoff + 3 * 1024, ZROW - 3 * 1024)])
    plsc.subcore_barrier()

    start, nch = _chunk_range(w)
    iota = lax.iota(jnp.int32, L)

    def chunk(ci, carry):
        pltpu.sync_copy(src2.at[start + ci], srcb)
        pltpu.sync_copy(dst2.at[start + ci], dstb)
        pltpu.sync_copy(typ2.at[start + ci], typb)
        for j in range(CHR):
            for g in range(128 // L):
                sl = pl.ds(g * L, L)
                gidx[j, sl] = typb[j, sl] * am + srcb[j, sl] * bm
        for j in range(CHR):
            pltpu.sync_copy(table.at[gidx.at[j]], rows.at[j])
            pltpu.sync_copy(inv2.at[dstb.at[j]], normb.at[j])
        for j in range(CHR):
            jv = jnp.full((L,), j, jnp.int32)

            def edge(e, ecarry):
                ev = jnp.full((L,), e, jnp.int32)
                t16 = plsc.load_gather(typb, [jv, ev])
                n16 = plsc.load_gather(normb, [jv, ev, t16])
                r16 = plsc.load_gather(rows, [jv, ev, iota])
                plsc.store_scatter(rows, [jv, ev, iota], r16 * n16)
                return ecarry

            lax.fori_loop(0, 128, edge, 0, unroll=4)
        for j in range(CHR):
            pltpu.sync_copy(rows.at[j], acc.at[dstb.at[j]], add=True)
        return carry

    lax.fori_loop(0, nch, chunk, 0)
    plsc.subcore_barrier()
    for k in range(3):
        pltpu.sync_copy(acc.at[pl.ds(zoff + k * 1024, 1024)], zbuf)
        pltpu.sync_copy(zbuf, out.at[c, pl.ds(zoff + k * 1024, 1024)])
    ztail = ZROW - 3 * 1024
    pltpu.sync_copy(acc.at[pl.ds(zoff + 3 * 1024, ztail)],
                    zbuf.at[pl.ds(0, ztail)])
    pltpu.sync_copy(zbuf.at[pl.ds(0, ztail)],
                    out.at[c, pl.ds(zoff + 3 * 1024, ztail)])


def _make_gss(am, bm):
    return functools.partial(
        pl.kernel,
        out_type=jax.ShapeDtypeStruct((NC, N, H), jnp.float32),
        mesh=_MESH,
        compiler_params=_SC_PARAMS,
        scratch_types=[
            pltpu.VMEM((CHR, 128), jnp.int32),
            pltpu.VMEM((CHR, 128), jnp.int32),
            pltpu.VMEM((CHR, 128), jnp.int32),
            pltpu.VMEM((CHR, 128), jnp.int32),
            pltpu.VMEM((CHR, 128, L), jnp.float32),
            pltpu.VMEM((CHR, 128, L), jnp.float32),
            pltpu.VMEM((1024, H), jnp.float32),
            pltpu.VMEM_SHARED((N, H), jnp.float32),
        ],
    )(functools.partial(_gss_body, am, bm))


_gss_l1 = _make_gss(N, 1)
_gss_l2 = _make_gss(1, R)


_BK1 = 32000


def _w1tab_body(comp_ref, basis_ref, out_ref):
    out_ref[...] = jnp.dot(comp_ref[...], basis_ref[...],
                           preferred_element_type=jnp.float32)


def _w1tab(comp1, basis_flat):
    return pl.pallas_call(
        _w1tab_body,
        grid=(800000 // _BK1,),
        in_specs=[
            pl.BlockSpec((R, NB), lambda i: (0, 0)),
            pl.BlockSpec((NB, _BK1), lambda i: (0, i)),
        ],
        out_specs=pl.BlockSpec((R, _BK1), lambda i: (0, i)),
        out_shape=jax.ShapeDtypeStruct((R, 800000), jnp.float32),
    )(comp1, basis_flat)


_BNI = 2000


def _inv_body(cnt_ref, out_ref):
    c = cnt_ref[0] + cnt_ref[1]
    out_ref[...] = 1.0 / jnp.maximum(c, 1.0)


def _inv(cnt):
    return pl.pallas_call(
        _inv_body,
        grid=(N // _BNI,),
        in_specs=[pl.BlockSpec((NC, _BNI, R), lambda i: (0, i, 0))],
        out_specs=pl.BlockSpec((_BNI, R), lambda i: (i, 0)),
        out_shape=jax.ShapeDtypeStruct((N, R), jnp.float32),
    )(cnt.reshape(NC, N, R))


_BN = 2000


def _xw_body(xacc_ref, root1_ref, b1_ref, w2m_ref, root2_ref, xw_ref, xr_ref):
    x = xacc_ref[0] + xacc_ref[1] + root1_ref[...] + b1_ref[...]
    xw_ref[...] = jnp.dot(x, w2m_ref[...], preferred_element_type=jnp.float32)
    xr_ref[...] = jnp.dot(x, root2_ref[...], preferred_element_type=jnp.float32)


def _xw(xacc, root1, b1, w2m, root2):
    return pl.pallas_call(
        _xw_body,
        grid=(N // _BN,),
        in_specs=[
            pl.BlockSpec((NC, _BN, H), lambda i: (0, i, 0)),
            pl.BlockSpec((_BN, H), lambda i: (i, 0)),
            pl.BlockSpec((1, H), lambda i: (0, 0)),
            pl.BlockSpec((H, R * C), lambda i: (0, 0)),
            pl.BlockSpec((H, C), lambda i: (0, 0)),
        ],
        out_specs=[
            pl.BlockSpec((_BN, R * C), lambda i: (i, 0)),
            pl.BlockSpec((_BN, C), lambda i: (i, 0)),
        ],
        out_shape=[
            jax.ShapeDtypeStruct((N, R * C), jnp.float32),
            jax.ShapeDtypeStruct((N, C), jnp.float32),
        ],
    )(xacc, root1, b1, w2m, root2)


def _fin_body(oacc_ref, xr_ref, b2_ref, out_ref):
    out_ref[...] = oacc_ref[0] + oacc_ref[1] + xr_ref[...] + b2_ref[...]


def _fin(oacc, xr, b2):
    return pl.pallas_call(
        _fin_body,
        grid=(N // _BN,),
        in_specs=[
            pl.BlockSpec((NC, _BN, C), lambda i: (0, i, 0)),
            pl.BlockSpec((_BN, C), lambda i: (i, 0)),
            pl.BlockSpec((1, C), lambda i: (0, 0)),
        ],
        out_specs=pl.BlockSpec((_BN, C), lambda i: (i, 0)),
        out_shape=jax.ShapeDtypeStruct((N, C), jnp.float32),
    )(oacc, xr, b2)


def kernel(edge_index, edge_type, basis1, comp1, root1, bias1,
           basis2, comp2, root2, bias2):
    src2 = edge_index[0].reshape(NCHUNK, CHR, 128)
    dst2 = edge_index[1].reshape(NCHUNK, CHR, 128)
    typ2 = edge_type.reshape(NCHUNK, CHR, 128)
    zc = jnp.zeros((1024,), jnp.float32)
    zn = jnp.zeros((1024, H), jnp.float32)

    cnt = _cnt_kernel(dst2, typ2, zc)
    inv = _inv(cnt)
    inv2 = jnp.pad(inv, ((0, 0), (0, L - R)))

    w1tab = _w1tab(comp1, basis1.reshape(NB, N * H)).reshape(R * N, H)
    xacc = _gss_l1(src2, dst2, typ2, w1tab, inv2, zn)

    w2 = jnp.einsum('rb,bio->rio', comp2, basis2)
    w2m = jnp.transpose(w2, (1, 0, 2)).reshape(H, R * C)
    xw_flat, xr = _xw(xacc, root1, bias1.reshape(1, H), w2m, root2)
    xw = xw_flat.reshape(N * R, C)

    oacc = _gss_l2(src2, dst2, typ2, xw, inv2, zn)
    return _fin(oacc, xr, bias2.reshape(1, C))


# speedup vs baseline: 26.7802x; 1.6567x over previous
import functools

import jax
import jax.numpy as jnp
from jax import lax
from jax.experimental import pallas as pl
from jax.experimental.pallas import tpu as pltpu, tpu_sc as plsc

N = 50000
E = 1600000
R = 8
NB = 30
H = 16
C = 16

NC = 2
NS = 16
L = 16
NW = NC * NS

EROWS = E // 128
CHR = 4
NCHUNK = EROWS // CHR
CH_BASE = NCHUNK // NW
CH_EXTRA = NCHUNK - CH_BASE * NW

ZROW = 3128
CROW_T = (N * R) // NS

_MESH = plsc.VectorSubcoreMesh(core_axis_name="c", subcore_axis_name="s")
_SC_PARAMS = pltpu.CompilerParams(needs_layout_passes=False, use_tc_tiling_on_sc=False)


def _wid():
    return lax.axis_index("s") * NC + lax.axis_index("c")


def _chunk_range(w):
    nch = CH_BASE + (w < CH_EXTRA).astype(jnp.int32)
    start = w * CH_BASE + jnp.minimum(w, CH_EXTRA)
    return start, nch


def _cnt_body(dst2, typ2, zc, out, dstb, typb, kidx, onesb, czbuf,
              sem_e, sem_s, cacc):
    c = lax.axis_index("c")
    s = lax.axis_index("s")
    w = _wid()
    pltpu.sync_copy(zc, czbuf)
    cbase = s * CROW_T
    for k in range(24):
        pltpu.sync_copy(czbuf, cacc.at[pl.ds(cbase + k * 1024, 1024)])
    pltpu.sync_copy(czbuf.at[pl.ds(0, CROW_T - 24 * 1024)],
                    cacc.at[pl.ds(cbase + 24 * 1024, CROW_T - 24 * 1024)])
    for j in range(CHR):
        for g in range(128 // L):
            onesb[j, pl.ds(g * L, L)] = jnp.full((L,), 1.0, jnp.float32)
    plsc.subcore_barrier()

    start, nch = _chunk_range(w)

    def chunk(ci, carry):
        h1 = pltpu.async_copy(dst2.at[start + ci], dstb, sem_e)
        h2 = pltpu.async_copy(typ2.at[start + ci], typb, sem_e)
        h1.wait()
        h2.wait()
        for j in range(CHR):
            for g in range(128 // L):
                sl = pl.ds(g * L, L)
                kidx[j, sl] = dstb[j, sl] * R + typb[j, sl]
        hs = [pltpu.async_copy(onesb.at[j], cacc.at[kidx.at[j]], sem_s,
                               add=True)
              for j in range(CHR)]
        for h in hs:
            h.wait()
        return carry

    lax.fori_loop(0, nch, chunk, 0)
    plsc.subcore_barrier()
    obase = c * (N * R) + s * CROW_T
    for k in range(24):
        pltpu.sync_copy(cacc.at[pl.ds(cbase + k * 1024, 1024)], czbuf)
        pltpu.sync_copy(czbuf, out.at[pl.ds(obase + k * 1024, 1024)])
    tail = CROW_T - 24 * 1024
    pltpu.sync_copy(cacc.at[pl.ds(cbase + 24 * 1024, tail)],
                    czbuf.at[pl.ds(0, tail)])
    pltpu.sync_copy(czbuf.at[pl.ds(0, tail)],
                    out.at[pl.ds(obase + 24 * 1024, tail)])


_cnt_kernel = functools.partial(
    pl.kernel,
    out_type=jax.ShapeDtypeStruct((NC * N * R,), jnp.float32),
    mesh=_MESH,
    compiler_params=_SC_PARAMS,
    scratch_types=[
        pltpu.VMEM((CHR, 128), jnp.int32),
        pltpu.VMEM((CHR, 128), jnp.int32),
        pltpu.VMEM((CHR, 128), jnp.int32),
        pltpu.VMEM((CHR, 128), jnp.float32),
        pltpu.VMEM((1024,), jnp.float32),
        pltpu.SemaphoreType.DMA,
        pltpu.SemaphoreType.DMA,
        pltpu.VMEM_SHARED((N * R,), jnp.float32),
    ],
)(_cnt_body)


def _gss_body(am, bm, src2, dst2, typ2, table, inv2, zn, out,
              srcb, dstb, typb, gidx, rows, normb, normc, zbuf,
              sem_e, sem_g, sem_s, acc):
    c = lax.axis_index("c")
    s = lax.axis_index("s")
    w = _wid()
    pltpu.sync_copy(zn, zbuf)
    zoff = jnp.minimum(s * ZROW, N - ZROW)
    for k in range(3):
        pltpu.sync_copy(zbuf, acc.at[pl.ds(zoff + k * 1024, 1024)])
    pltpu.sync_copy(zbuf.at[pl.ds(0, ZROW - 3 * 1024)],
                    acc.at[pl.ds(zoff + 3 * 1024, ZROW - 3 * 1024)])
    plsc.subcore_barrier()

    start, nch = _chunk_range(w)
    iota = lax.iota(jnp.int32, L)

    def chunk(ci, carry):
        h1 = pltpu.async_copy(src2.at[start + ci], srcb, sem_e)
        h2 = pltpu.async_copy(dst2.at[start + ci], dstb, sem_e)
        h3 = pltpu.async_copy(typ2.at[start + ci], typb, sem_e)
        h1.wait()
        h2.wait()
        h3.wait()
        gh = [pltpu.async_copy(inv2.at[dstb.at[j]], normb.at[j], sem_g)
              for j in range(CHR)]
        for j in range(CHR):
            for g in range(128 // L):
                sl = pl.ds(g * L, L)
                gidx[j, sl] = typb[j, sl] * am + srcb[j, sl] * bm
        gh += [pltpu.async_copy(table.at[gidx.at[j]], rows.at[j], sem_g)
               for j in range(CHR)]
        for h in gh:
            h.wait()
        for j in range(CHR):
            jv = jnp.full((L,), j, jnp.int32)
            for g in range(128 // L):
                sl = pl.ds(g * L, L)
                n16 = plsc.load_gather(normb, [jv, iota + (g * L),
                                               typb[j, sl]])
                normc[j, sl] = n16
        for j in range(CHR):
            jv = jnp.full((L,), j, jnp.int32)

            def edge(e, ecarry):
                ev = jnp.full((L,), e, jnp.int32)
                nb = plsc.load_gather(normc, [jv, ev])
                r16 = plsc.load_gather(rows, [jv, ev, iota])
                plsc.store_scatter(rows, [jv, ev, iota], r16 * nb)
                return ecarry

            lax.fori_loop(0, 128, edge, 0, unroll=8)
        sh = [pltpu.async_copy(rows.at[j], acc.at[dstb.at[j]], sem_s,
                               add=True)
              for j in range(CHR)]
        for h in sh:
            h.wait()
        return carry

    lax.fori_loop(0, nch, chunk, 0)
    plsc.subcore_barrier()
    for k in range(3):
        pltpu.sync_copy(acc.at[pl.ds(zoff + k * 1024, 1024)], zbuf)
        pltpu.sync_copy(zbuf, out.at[c, pl.ds(zoff + k * 1024, 1024)])
    ztail = ZROW - 3 * 1024
    pltpu.sync_copy(acc.at[pl.ds(zoff + 3 * 1024, ztail)],
                    zbuf.at[pl.ds(0, ztail)])
    pltpu.sync_copy(zbuf.at[pl.ds(0, ztail)],
                    out.at[c, pl.ds(zoff + 3 * 1024, ztail)])


def _make_gss(am, bm):
    return functools.partial(
        pl.kernel,
        out_type=jax.ShapeDtypeStruct((NC, N, H), jnp.float32),
        mesh=_MESH,
        compiler_params=_SC_PARAMS,
        scratch_types=[
            pltpu.VMEM((CHR, 128), jnp.int32),
            pltpu.VMEM((CHR, 128), jnp.int32),
            pltpu.VMEM((CHR, 128), jnp.int32),
            pltpu.VMEM((CHR, 128), jnp.int32),
            pltpu.VMEM((CHR, 128, L), jnp.float32),
            pltpu.VMEM((CHR, 128, L), jnp.float32),
            pltpu.VMEM((CHR, 128), jnp.float32),
            pltpu.VMEM((1024, H), jnp.float32),
            pltpu.SemaphoreType.DMA,
            pltpu.SemaphoreType.DMA,
            pltpu.SemaphoreType.DMA,
            pltpu.VMEM_SHARED((N, H), jnp.float32),
        ],
    )(functools.partial(_gss_body, am, bm))


_gss_l1 = _make_gss(N, 1)
_gss_l2 = _make_gss(1, R)


_BK1 = 32000


def _w1tab_body(comp_ref, basis_ref, out_ref):
    out_ref[...] = jnp.dot(comp_ref[...], basis_ref[...],
                           preferred_element_type=jnp.float32)


def _w1tab(comp1, basis_flat):
    return pl.pallas_call(
        _w1tab_body,
        grid=(800000 // _BK1,),
        in_specs=[
            pl.BlockSpec((R, NB), lambda i: (0, 0)),
            pl.BlockSpec((NB, _BK1), lambda i: (0, i)),
        ],
        out_specs=pl.BlockSpec((R, _BK1), lambda i: (0, i)),
        out_shape=jax.ShapeDtypeStruct((R, 800000), jnp.float32),
    )(comp1, basis_flat)


_BNI = 2000


def _inv_body(cnt_ref, out_ref):
    c = cnt_ref[0] + cnt_ref[1]
    out_ref[...] = 1.0 / jnp.maximum(c, 1.0)


def _inv(cnt):
    return pl.pallas_call(
        _inv_body,
        grid=(N // _BNI,),
        in_specs=[pl.BlockSpec((NC, _BNI, R), lambda i: (0, i, 0))],
        out_specs=pl.BlockSpec((_BNI, R), lambda i: (i, 0)),
        out_shape=jax.ShapeDtypeStruct((N, R), jnp.float32),
    )(cnt.reshape(NC, N, R))


_BN = 2000


def _xw_body(xacc_ref, root1_ref, b1_ref, w2m_ref, root2_ref, xw_ref, xr_ref):
    x = xacc_ref[0] + xacc_ref[1] + root1_ref[...] + b1_ref[...]
    xw_ref[...] = jnp.dot(x, w2m_ref[...], preferred_element_type=jnp.float32)
    xr_ref[...] = jnp.dot(x, root2_ref[...], preferred_element_type=jnp.float32)


def _xw(xacc, root1, b1, w2m, root2):
    return pl.pallas_call(
        _xw_body,
        grid=(N // _BN,),
        in_specs=[
            pl.BlockSpec((NC, _BN, H), lambda i: (0, i, 0)),
            pl.BlockSpec((_BN, H), lambda i: (i, 0)),
            pl.BlockSpec((1, H), lambda i: (0, 0)),
            pl.BlockSpec((H, R * C), lambda i: (0, 0)),
            pl.BlockSpec((H, C), lambda i: (0, 0)),
        ],
        out_specs=[
            pl.BlockSpec((_BN, R * C), lambda i: (i, 0)),
            pl.BlockSpec((_BN, C), lambda i: (i, 0)),
        ],
        out_shape=[
            jax.ShapeDtypeStruct((N, R * C), jnp.float32),
            jax.ShapeDtypeStruct((N, C), jnp.float32),
        ],
    )(xacc, root1, b1, w2m, root2)


def _fin_body(oacc_ref, xr_ref, b2_ref, out_ref):
    out_ref[...] = oacc_ref[0] + oacc_ref[1] + xr_ref[...] + b2_ref[...]


def _fin(oacc, xr, b2):
    return pl.pallas_call(
        _fin_body,
        grid=(N // _BN,),
        in_specs=[
            pl.BlockSpec((NC, _BN, C), lambda i: (0, i, 0)),
            pl.BlockSpec((_BN, C), lambda i: (i, 0)),
            pl.BlockSpec((1, C), lambda i: (0, 0)),
        ],
        out_specs=pl.BlockSpec((_BN, C), lambda i: (i, 0)),
        out_shape=jax.ShapeDtypeStruct((N, C), jnp.float32),
    )(oacc, xr, b2)


def kernel(edge_index, edge_type, basis1, comp1, root1, bias1,
           basis2, comp2, root2, bias2):
    src2 = edge_index[0].reshape(NCHUNK, CHR, 128)
    dst2 = edge_index[1].reshape(NCHUNK, CHR, 128)
    typ2 = edge_type.reshape(NCHUNK, CHR, 128)
    zc = jnp.zeros((1024,), jnp.float32)
    zn = jnp.zeros((1024, H), jnp.float32)

    cnt = _cnt_kernel(dst2, typ2, zc)
    inv = _inv(cnt)
    inv2 = jnp.pad(inv, ((0, 0), (0, L - R)))

    w1tab = _w1tab(comp1, basis1.reshape(NB, N * H)).reshape(R * N, H)
    xacc = _gss_l1(src2, dst2, typ2, w1tab, inv2, zn)

    w2 = jnp.einsum('rb,bio->rio', comp2, basis2)
    w2m = jnp.transpose(w2, (1, 0, 2)).reshape(H, R * C)
    xw_flat, xr = _xw(xacc, root1, bias1.reshape(1, H), w2m, root2)
    xw = xw_flat.reshape(N * R, C)

    oacc = _gss_l2(src2, dst2, typ2, xw, inv2, zn)
    return _fin(oacc, xr, bias2.reshape(1, C))


# speedup vs baseline: 29.4994x; 1.1015x over previous
import functools

import jax
import jax.numpy as jnp
from jax import lax
from jax.experimental import pallas as pl
from jax.experimental.pallas import tpu as pltpu, tpu_sc as plsc

N = 50000
E = 1600000
R = 8
NB = 30
H = 16
C = 16

NC = 2
NS = 16
L = 16
NW = NC * NS

EROWS = E // 128
CHR = 4
NCHUNK = EROWS // CHR
CH_BASE = NCHUNK // NW
CH_EXTRA = NCHUNK - CH_BASE * NW

ZROW = 3128
CROW_T = (N * R) // NS

_MESH = plsc.VectorSubcoreMesh(core_axis_name="c", subcore_axis_name="s")
_SC_PARAMS = pltpu.CompilerParams(needs_layout_passes=False, use_tc_tiling_on_sc=False)


def _wid():
    return lax.axis_index("s") * NC + lax.axis_index("c")


def _chunk_range(w):
    nch = CH_BASE + (w < CH_EXTRA).astype(jnp.int32)
    start = w * CH_BASE + jnp.minimum(w, CH_EXTRA)
    return start, nch


def _cnt_body(dst2, typ2, zc, out, dstb, typb, kidx, onesb, czbuf,
              sem_e, sem_s, cacc):
    c = lax.axis_index("c")
    s = lax.axis_index("s")
    w = _wid()
    pltpu.sync_copy(zc, czbuf)
    cbase = s * CROW_T
    for k in range(24):
        pltpu.sync_copy(czbuf, cacc.at[pl.ds(cbase + k * 1024, 1024)])
    pltpu.sync_copy(czbuf.at[pl.ds(0, CROW_T - 24 * 1024)],
                    cacc.at[pl.ds(cbase + 24 * 1024, CROW_T - 24 * 1024)])
    for j in range(CHR):
        for g in range(128 // L):
            onesb[j, pl.ds(g * L, L)] = jnp.full((L,), 1.0, jnp.float32)
    plsc.subcore_barrier()

    start, nch = _chunk_range(w)

    def chunk(ci, carry):
        h1 = pltpu.async_copy(dst2.at[start + ci], dstb, sem_e)
        h2 = pltpu.async_copy(typ2.at[start + ci], typb, sem_e)
        h1.wait()
        h2.wait()
        for j in range(CHR):
            for g in range(128 // L):
                sl = pl.ds(g * L, L)
                kidx[j, sl] = dstb[j, sl] * R + typb[j, sl]
        hs = [pltpu.async_copy(onesb.at[j], cacc.at[kidx.at[j]], sem_s,
                               add=True)
              for j in range(CHR)]
        for h in hs:
            h.wait()
        return carry

    lax.fori_loop(0, nch, chunk, 0)
    plsc.subcore_barrier()
    obase = c * (N * R) + s * CROW_T
    for k in range(24):
        pltpu.sync_copy(cacc.at[pl.ds(cbase + k * 1024, 1024)], czbuf)
        pltpu.sync_copy(czbuf, out.at[pl.ds(obase + k * 1024, 1024)])
    tail = CROW_T - 24 * 1024
    pltpu.sync_copy(cacc.at[pl.ds(cbase + 24 * 1024, tail)],
                    czbuf.at[pl.ds(0, tail)])
    pltpu.sync_copy(czbuf.at[pl.ds(0, tail)],
                    out.at[pl.ds(obase + 24 * 1024, tail)])


_cnt_kernel = functools.partial(
    pl.kernel,
    out_type=jax.ShapeDtypeStruct((NC * N * R,), jnp.float32),
    mesh=_MESH,
    compiler_params=_SC_PARAMS,
    scratch_types=[
        pltpu.VMEM((CHR, 128), jnp.int32),
        pltpu.VMEM((CHR, 128), jnp.int32),
        pltpu.VMEM((CHR, 128), jnp.int32),
        pltpu.VMEM((CHR, 128), jnp.float32),
        pltpu.VMEM((1024,), jnp.float32),
        pltpu.SemaphoreType.DMA,
        pltpu.SemaphoreType.DMA,
        pltpu.VMEM_SHARED((N * R,), jnp.float32),
    ],
)(_cnt_body)


def _gss_body(am, bm, src2, dst2, typ2, table, inv2, zn, out,
              srcb, dstb, typb, gidx, rows, normb, normc, zbuf,
              sem_ea, sem_eb, sem_ga, sem_gb, sem_sa, sem_sb, acc):
    c = lax.axis_index("c")
    s = lax.axis_index("s")
    w = _wid()
    pltpu.sync_copy(zn, zbuf)
    zoff = jnp.minimum(s * ZROW, N - ZROW)
    for k in range(3):
        pltpu.sync_copy(zbuf, acc.at[pl.ds(zoff + k * 1024, 1024)])
    pltpu.sync_copy(zbuf.at[pl.ds(0, ZROW - 3 * 1024)],
                    acc.at[pl.ds(zoff + 3 * 1024, ZROW - 3 * 1024)])
    plsc.subcore_barrier()

    start, nch = _chunk_range(w)
    iota = lax.iota(jnp.int32, L)

    def _loads(ci, b, sem):
        return [pltpu.async_copy(src2.at[ci], srcb.at[b], sem),
                pltpu.async_copy(dst2.at[ci], dstb.at[b], sem),
                pltpu.async_copy(typ2.at[ci], typb.at[b], sem)]

    def _fire_gathers(b, sem):
        hs = [pltpu.async_copy(inv2.at[dstb.at[b, j]], normb.at[b, j], sem)
              for j in range(CHR)]
        for j in range(CHR):
            for g in range(128 // L):
                sl = pl.ds(g * L, L)
                gidx[b, j, sl] = typb[b, j, sl] * am + srcb[b, j, sl] * bm
        hs += [pltpu.async_copy(table.at[gidx.at[b, j]], rows.at[b, j], sem)
               for j in range(CHR)]
        return hs

    def _scale(b):
        bv = jnp.full((L,), b, jnp.int32)
        for j in range(CHR):
            jv = jnp.full((L,), j, jnp.int32)
            for g in range(128 // L):
                sl = pl.ds(g * L, L)
                n16 = plsc.load_gather(normb, [bv, jv, iota + (g * L),
                                               typb[b, j, sl]])
                normc[b, j, sl] = n16

            def edge(e, ecarry):
                ev = jnp.full((L,), e, jnp.int32)
                nb = plsc.load_gather(normc, [bv, jv, ev])
                rows[b, j, e] = rows[b, j, e] * nb
                return ecarry

            lax.fori_loop(0, 128, edge, 0, unroll=8)

    def _fire_scatters(b, sem):
        return [pltpu.async_copy(rows.at[b, j], acc.at[dstb.at[b, j]], sem,
                                 add=True)
                for j in range(CHR)]

    def pair(pi, carry):
        ci_a = start + 2 * pi
        e_a = _loads(ci_a, 0, sem_ea)
        e_b = _loads(ci_a + 1, 1, sem_eb)
        for h in e_a:
            h.wait()
        g_a = _fire_gathers(0, sem_ga)
        for h in e_b:
            h.wait()
        g_b = _fire_gathers(1, sem_gb)
        for h in g_a:
            h.wait()
        _scale(0)
        s_a = _fire_scatters(0, sem_sa)
        for h in g_b:
            h.wait()
        _scale(1)
        s_b = _fire_scatters(1, sem_sb)
        for h in s_a:
            h.wait()
        for h in s_b:
            h.wait()
        return carry

    lax.fori_loop(0, nch // 2, pair, 0)

    @pl.when(nch % 2 == 1)
    def _tail_chunk():
        ci = start + nch - 1
        e0 = _loads(ci, 0, sem_ea)
        for h in e0:
            h.wait()
        g0 = _fire_gathers(0, sem_ga)
        for h in g0:
            h.wait()
        _scale(0)
        s0 = _fire_scatters(0, sem_sa)
        for h in s0:
            h.wait()
    plsc.subcore_barrier()
    for k in range(3):
        pltpu.sync_copy(acc.at[pl.ds(zoff + k * 1024, 1024)], zbuf)
        pltpu.sync_copy(zbuf, out.at[c, pl.ds(zoff + k * 1024, 1024)])
    ztail = ZROW - 3 * 1024
    pltpu.sync_copy(acc.at[pl.ds(zoff + 3 * 1024, ztail)],
                    zbuf.at[pl.ds(0, ztail)])
    pltpu.sync_copy(zbuf.at[pl.ds(0, ztail)],
                    out.at[c, pl.ds(zoff + 3 * 1024, ztail)])


def _make_gss(am, bm):
    return functools.partial(
        pl.kernel,
        out_type=jax.ShapeDtypeStruct((NC, N, H), jnp.float32),
        mesh=_MESH,
        compiler_params=_SC_PARAMS,
        scratch_types=[
            pltpu.VMEM((2, CHR, 128), jnp.int32),
            pltpu.VMEM((2, CHR, 128), jnp.int32),
            pltpu.VMEM((2, CHR, 128), jnp.int32),
            pltpu.VMEM((2, CHR, 128), jnp.int32),
            pltpu.VMEM((2, CHR, 128, L), jnp.float32),
            pltpu.VMEM((2, CHR, 128, L), jnp.float32),
            pltpu.VMEM((2, CHR, 128), jnp.float32),
            pltpu.VMEM((1024, H), jnp.float32),
            pltpu.SemaphoreType.DMA,
            pltpu.SemaphoreType.DMA,
            pltpu.SemaphoreType.DMA,
            pltpu.SemaphoreType.DMA,
            pltpu.SemaphoreType.DMA,
            pltpu.SemaphoreType.DMA,
            pltpu.VMEM_SHARED((N, H), jnp.float32),
        ],
    )(functools.partial(_gss_body, am, bm))


_gss_l1 = _make_gss(N, 1)
_gss_l2 = _make_gss(1, R)


_BK1 = 32000


def _w1tab_body(comp_ref, basis_ref, out_ref):
    out_ref[...] = jnp.dot(comp_ref[...], basis_ref[...],
                           preferred_element_type=jnp.float32)


def _w1tab(comp1, basis_flat):
    return pl.pallas_call(
        _w1tab_body,
        grid=(800000 // _BK1,),
        in_specs=[
            pl.BlockSpec((R, NB), lambda i: (0, 0)),
            pl.BlockSpec((NB, _BK1), lambda i: (0, i)),
        ],
        out_specs=pl.BlockSpec((R, _BK1), lambda i: (0, i)),
        out_shape=jax.ShapeDtypeStruct((R, 800000), jnp.float32),
    )(comp1, basis_flat)


_BNI = 2000


def _inv_body(cnt_ref, out_ref):
    c = cnt_ref[0] + cnt_ref[1]
    out_ref[...] = 1.0 / jnp.maximum(c, 1.0)


def _inv(cnt):
    return pl.pallas_call(
        _inv_body,
        grid=(N // _BNI,),
        in_specs=[pl.BlockSpec((NC, _BNI, R), lambda i: (0, i, 0))],
        out_specs=pl.BlockSpec((_BNI, R), lambda i: (i, 0)),
        out_shape=jax.ShapeDtypeStruct((N, R), jnp.float32),
    )(cnt.reshape(NC, N, R))


_BN = 2000


def _xw_body(xacc_ref, root1_ref, b1_ref, w2m_ref, root2_ref, xw_ref, xr_ref):
    x = xacc_ref[0] + xacc_ref[1] + root1_ref[...] + b1_ref[...]
    xw_ref[...] = jnp.dot(x, w2m_ref[...], preferred_element_type=jnp.float32)
    xr_ref[...] = jnp.dot(x, root2_ref[...], preferred_element_type=jnp.float32)


def _xw(xacc, root1, b1, w2m, root2):
    return pl.pallas_call(
        _xw_body,
        grid=(N // _BN,),
        in_specs=[
            pl.BlockSpec((NC, _BN, H), lambda i: (0, i, 0)),
            pl.BlockSpec((_BN, H), lambda i: (i, 0)),
            pl.BlockSpec((1, H), lambda i: (0, 0)),
            pl.BlockSpec((H, R * C), lambda i: (0, 0)),
            pl.BlockSpec((H, C), lambda i: (0, 0)),
        ],
        out_specs=[
            pl.BlockSpec((_BN, R * C), lambda i: (i, 0)),
            pl.BlockSpec((_BN, C), lambda i: (i, 0)),
        ],
        out_shape=[
            jax.ShapeDtypeStruct((N, R * C), jnp.float32),
            jax.ShapeDtypeStruct((N, C), jnp.float32),
        ],
    )(xacc, root1, b1, w2m, root2)


def _fin_body(oacc_ref, xr_ref, b2_ref, out_ref):
    out_ref[...] = oacc_ref[0] + oacc_ref[1] + xr_ref[...] + b2_ref[...]


def _fin(oacc, xr, b2):
    return pl.pallas_call(
        _fin_body,
        grid=(N // _BN,),
        in_specs=[
            pl.BlockSpec((NC, _BN, C), lambda i: (0, i, 0)),
            pl.BlockSpec((_BN, C), lambda i: (i, 0)),
            pl.BlockSpec((1, C), lambda i: (0, 0)),
        ],
        out_specs=pl.BlockSpec((_BN, C), lambda i: (i, 0)),
        out_shape=jax.ShapeDtypeStruct((N, C), jnp.float32),
    )(oacc, xr, b2)


def kernel(edge_index, edge_type, basis1, comp1, root1, bias1,
           basis2, comp2, root2, bias2):
    src2 = edge_index[0].reshape(NCHUNK, CHR, 128)
    dst2 = edge_index[1].reshape(NCHUNK, CHR, 128)
    typ2 = edge_type.reshape(NCHUNK, CHR, 128)
    zc = jnp.zeros((1024,), jnp.float32)
    zn = jnp.zeros((1024, H), jnp.float32)

    cnt = _cnt_kernel(dst2, typ2, zc)
    inv = _inv(cnt)
    inv2 = jnp.pad(inv, ((0, 0), (0, L - R)))

    w1tab = _w1tab(comp1, basis1.reshape(NB, N * H)).reshape(R * N, H)
    xacc = _gss_l1(src2, dst2, typ2, w1tab, inv2, zn)

    w2 = jnp.einsum('rb,bio->rio', comp2, basis2)
    w2m = jnp.transpose(w2, (1, 0, 2)).reshape(H, R * C)
    xw_flat, xr = _xw(xacc, root1, bias1.reshape(1, H), w2m, root2)
    xw = xw_flat.reshape(N * R, C)

    oacc = _gss_l2(src2, dst2, typ2, xw, inv2, zn)
    return _fin(oacc, xr, bias2.reshape(1, C))


# speedup vs baseline: 53.0527x; 1.7984x over previous
import functools

import jax
import jax.numpy as jnp
from jax import lax
from jax.experimental import pallas as pl
from jax.experimental.pallas import tpu as pltpu, tpu_sc as plsc

N = 50000
E = 1600000
R = 8
NB = 30
H = 16
C = 16

NC = 2
NS = 16
L = 16
NW = NC * NS

EROWS = E // 128
CHR = 4
NCHUNK = EROWS // CHR
CH_BASE = NCHUNK // NW
CH_EXTRA = NCHUNK - CH_BASE * NW

ZROW = 3128
CROW_T = (N * R) // NS

_MESH = plsc.VectorSubcoreMesh(core_axis_name="c", subcore_axis_name="s")
_SC_PARAMS = pltpu.CompilerParams(needs_layout_passes=False, use_tc_tiling_on_sc=False)


def _wid():
    return lax.axis_index("s") * NC + lax.axis_index("c")


def _chunk_range(w):
    nch = CH_BASE + (w < CH_EXTRA).astype(jnp.int32)
    start = w * CH_BASE + jnp.minimum(w, CH_EXTRA)
    return start, nch


def _cnt_body(dst2, typ2, zc, out, dstb, typb, kidx, onesb, czbuf,
              sem_e, sem_s, cacc):
    c = lax.axis_index("c")
    s = lax.axis_index("s")
    w = _wid()
    pltpu.sync_copy(zc, czbuf)
    cbase = s * CROW_T
    for k in range(24):
        pltpu.sync_copy(czbuf, cacc.at[pl.ds(cbase + k * 1024, 1024)])
    pltpu.sync_copy(czbuf.at[pl.ds(0, CROW_T - 24 * 1024)],
                    cacc.at[pl.ds(cbase + 24 * 1024, CROW_T - 24 * 1024)])
    for j in range(CHR):
        for g in range(128 // L):
            onesb[j, pl.ds(g * L, L)] = jnp.full((L,), 1.0, jnp.float32)
    plsc.subcore_barrier()

    start, nch = _chunk_range(w)

    def chunk(ci, carry):
        h1 = pltpu.async_copy(dst2.at[start + ci], dstb, sem_e)
        h2 = pltpu.async_copy(typ2.at[start + ci], typb, sem_e)
        h1.wait()
        h2.wait()
        for j in range(CHR):
            for g in range(128 // L):
                sl = pl.ds(g * L, L)
                kidx[j, sl] = dstb[j, sl] * R + typb[j, sl]
        hs = [pltpu.async_copy(onesb.at[j], cacc.at[kidx.at[j]], sem_s,
                               add=True)
              for j in range(CHR)]
        for h in hs:
            h.wait()
        return carry

    lax.fori_loop(0, nch, chunk, 0)
    plsc.subcore_barrier()
    obase = c * (N * R) + s * CROW_T
    for k in range(24):
        pltpu.sync_copy(cacc.at[pl.ds(cbase + k * 1024, 1024)], czbuf)
        pltpu.sync_copy(czbuf, out.at[pl.ds(obase + k * 1024, 1024)])
    tail = CROW_T - 24 * 1024
    pltpu.sync_copy(cacc.at[pl.ds(cbase + 24 * 1024, tail)],
                    czbuf.at[pl.ds(0, tail)])
    pltpu.sync_copy(czbuf.at[pl.ds(0, tail)],
                    out.at[pl.ds(obase + 24 * 1024, tail)])


_cnt_kernel = functools.partial(
    pl.kernel,
    out_type=jax.ShapeDtypeStruct((NC * N * R,), jnp.float32),
    mesh=_MESH,
    compiler_params=_SC_PARAMS,
    scratch_types=[
        pltpu.VMEM((CHR, 128), jnp.int32),
        pltpu.VMEM((CHR, 128), jnp.int32),
        pltpu.VMEM((CHR, 128), jnp.int32),
        pltpu.VMEM((CHR, 128), jnp.float32),
        pltpu.VMEM((1024,), jnp.float32),
        pltpu.SemaphoreType.DMA,
        pltpu.SemaphoreType.DMA,
        pltpu.VMEM_SHARED((N * R,), jnp.float32),
    ],
)(_cnt_body)


def _gss_body(am, bm, src2, dst2, typ2, table, inv2, zn, out,
              srcb, dstb, typb, gidx, rows, normb, normc, zbuf,
              sem_ea, sem_eb, sem_ga, sem_gb, sem_sa, sem_sb, acc):
    c = lax.axis_index("c")
    s = lax.axis_index("s")
    w = _wid()
    pltpu.sync_copy(zn, zbuf)
    zoff = jnp.minimum(s * ZROW, N - ZROW)
    for k in range(3):
        pltpu.sync_copy(zbuf, acc.at[pl.ds(zoff + k * 1024, 1024)])
    pltpu.sync_copy(zbuf.at[pl.ds(0, ZROW - 3 * 1024)],
                    acc.at[pl.ds(zoff + 3 * 1024, ZROW - 3 * 1024)])
    plsc.subcore_barrier()

    start, nch = _chunk_range(w)
    iota = lax.iota(jnp.int32, L)

    def _loads(ci, b, sem):
        return [pltpu.async_copy(src2.at[ci], srcb.at[b], sem),
                pltpu.async_copy(dst2.at[ci], dstb.at[b], sem),
                pltpu.async_copy(typ2.at[ci], typb.at[b], sem)]

    def _fire_gathers(b, sem):
        hs = [pltpu.async_copy(inv2.at[dstb.at[b, j]], normb.at[b, j], sem)
              for j in range(CHR)]
        for j in range(CHR):
            for g in range(128 // L):
                sl = pl.ds(g * L, L)
                gidx[b, j, sl] = typb[b, j, sl] * am + srcb[b, j, sl] * bm
        hs += [pltpu.async_copy(table.at[gidx.at[b, j]], rows.at[b, j], sem)
               for j in range(CHR)]
        return hs

    def _scale(b):
        bv = jnp.full((L,), b, jnp.int32)
        for j in range(CHR):
            jv = jnp.full((L,), j, jnp.int32)
            for g in range(128 // L):
                sl = pl.ds(g * L, L)
                n16 = plsc.load_gather(normb, [bv, jv, iota + (g * L),
                                               typb[b, j, sl]])
                normc[b, j, sl] = n16

            def edge(e, ecarry):
                ev = jnp.full((L,), e, jnp.int32)
                nb = plsc.load_gather(normc, [bv, jv, ev])
                rows[b, j, e] = rows[b, j, e] * nb
                return ecarry

            lax.fori_loop(0, 128, edge, 0, unroll=8)

    def _fire_scatters(b, sem):
        return [pltpu.async_copy(rows.at[b, j], acc.at[dstb.at[b, j]], sem,
                                 add=True)
                for j in range(CHR)]

    def pair(pi, carry):
        ci_a = start + 2 * pi
        e_a = _loads(ci_a, 0, sem_ea)
        e_b = _loads(ci_a + 1, 1, sem_eb)
        for h in e_a:
            h.wait()
        g_a = _fire_gathers(0, sem_ga)
        for h in e_b:
            h.wait()
        g_b = _fire_gathers(1, sem_gb)
        for h in g_a:
            h.wait()
        _scale(0)
        s_a = _fire_scatters(0, sem_sa)
        for h in g_b:
            h.wait()
        _scale(1)
        s_b = _fire_scatters(1, sem_sb)
        for h in s_a:
            h.wait()
        for h in s_b:
            h.wait()
        return carry

    lax.fori_loop(0, nch // 2, pair, 0)

    @pl.when(nch % 2 == 1)
    def _tail_chunk():
        ci = start + nch - 1
        e0 = _loads(ci, 0, sem_ea)
        for h in e0:
            h.wait()
        g0 = _fire_gathers(0, sem_ga)
        for h in g0:
            h.wait()
        _scale(0)
        s0 = _fire_scatters(0, sem_sa)
        for h in s0:
            h.wait()
    plsc.subcore_barrier()
    for k in range(3):
        pltpu.sync_copy(acc.at[pl.ds(zoff + k * 1024, 1024)], zbuf)
        pltpu.sync_copy(zbuf, out.at[c, pl.ds(zoff + k * 1024, 1024)])
    ztail = ZROW - 3 * 1024
    pltpu.sync_copy(acc.at[pl.ds(zoff + 3 * 1024, ztail)],
                    zbuf.at[pl.ds(0, ztail)])
    pltpu.sync_copy(zbuf.at[pl.ds(0, ztail)],
                    out.at[c, pl.ds(zoff + 3 * 1024, ztail)])


def _make_gss(am, bm):
    return functools.partial(
        pl.kernel,
        out_type=jax.ShapeDtypeStruct((NC, N, H), jnp.float32),
        mesh=_MESH,
        compiler_params=_SC_PARAMS,
        scratch_types=[
            pltpu.VMEM((2, CHR, 128), jnp.int32),
            pltpu.VMEM((2, CHR, 128), jnp.int32),
            pltpu.VMEM((2, CHR, 128), jnp.int32),
            pltpu.VMEM((2, CHR, 128), jnp.int32),
            pltpu.VMEM((2, CHR, 128, L), jnp.float32),
            pltpu.VMEM((2, CHR, 128, L), jnp.float32),
            pltpu.VMEM((2, CHR, 128), jnp.float32),
            pltpu.VMEM((1024, H), jnp.float32),
            pltpu.SemaphoreType.DMA,
            pltpu.SemaphoreType.DMA,
            pltpu.SemaphoreType.DMA,
            pltpu.SemaphoreType.DMA,
            pltpu.SemaphoreType.DMA,
            pltpu.SemaphoreType.DMA,
            pltpu.VMEM_SHARED((N, H), jnp.float32),
        ],
    )(functools.partial(_gss_body, am, bm))


_gss = _make_gss(1, R)


_BN1 = 4096


def _w1tab_body(basis_ref, compe_ref, out_ref):
    out_ref[...] = lax.dot_general(
        basis_ref[...], compe_ref[...],
        dimension_numbers=(((0,), (1,)), ((), ())),
        preferred_element_type=jnp.float32)


def _w1tab(basis_m, comp_e):
    grid = (N + _BN1 - 1) // _BN1
    return pl.pallas_call(
        _w1tab_body,
        grid=(grid,),
        in_specs=[
            pl.BlockSpec((NB * H, _BN1), lambda i: (0, i)),
            pl.BlockSpec((R * H, NB * H), lambda i: (0, 0)),
        ],
        out_specs=pl.BlockSpec((_BN1, R * H), lambda i: (i, 0)),
        out_shape=jax.ShapeDtypeStruct((N, R * H), jnp.float32),
    )(basis_m, comp_e)


_BNI = 2000


def _inv_body(cnt_ref, out_ref):
    c = cnt_ref[0] + cnt_ref[1]
    out_ref[...] = 1.0 / jnp.maximum(c, 1.0)


def _inv(cnt):
    return pl.pallas_call(
        _inv_body,
        grid=(N // _BNI,),
        in_specs=[pl.BlockSpec((NC, _BNI, R), lambda i: (0, i, 0))],
        out_specs=pl.BlockSpec((_BNI, R), lambda i: (i, 0)),
        out_shape=jax.ShapeDtypeStruct((N, R), jnp.float32),
    )(cnt.reshape(NC, N, R))


_BN = 2000


def _xw_body(xacc_ref, root1_ref, b1_ref, w2m_ref, root2_ref, xw_ref, xr_ref):
    x = xacc_ref[0] + xacc_ref[1] + root1_ref[...] + b1_ref[...]
    xw_ref[...] = jnp.dot(x, w2m_ref[...], preferred_element_type=jnp.float32)
    xr_ref[...] = jnp.dot(x, root2_ref[...], preferred_element_type=jnp.float32)


def _xw(xacc, root1, b1, w2m, root2):
    return pl.pallas_call(
        _xw_body,
        grid=(N // _BN,),
        in_specs=[
            pl.BlockSpec((NC, _BN, H), lambda i: (0, i, 0)),
            pl.BlockSpec((_BN, H), lambda i: (i, 0)),
            pl.BlockSpec((1, H), lambda i: (0, 0)),
            pl.BlockSpec((H, R * C), lambda i: (0, 0)),
            pl.BlockSpec((H, C), lambda i: (0, 0)),
        ],
        out_specs=[
            pl.BlockSpec((_BN, R * C), lambda i: (i, 0)),
            pl.BlockSpec((_BN, C), lambda i: (i, 0)),
        ],
        out_shape=[
            jax.ShapeDtypeStruct((N, R * C), jnp.float32),
            jax.ShapeDtypeStruct((N, C), jnp.float32),
        ],
    )(xacc, root1, b1, w2m, root2)


def _fin_body(oacc_ref, xr_ref, b2_ref, out_ref):
    out_ref[...] = oacc_ref[0] + oacc_ref[1] + xr_ref[...] + b2_ref[...]


def _fin(oacc, xr, b2):
    return pl.pallas_call(
        _fin_body,
        grid=(N // _BN,),
        in_specs=[
            pl.BlockSpec((NC, _BN, C), lambda i: (0, i, 0)),
            pl.BlockSpec((_BN, C), lambda i: (i, 0)),
            pl.BlockSpec((1, C), lambda i: (0, 0)),
        ],
        out_specs=pl.BlockSpec((_BN, C), lambda i: (i, 0)),
        out_shape=jax.ShapeDtypeStruct((N, C), jnp.float32),
    )(oacc, xr, b2)


def kernel(edge_index, edge_type, basis1, comp1, root1, bias1,
           basis2, comp2, root2, bias2):
    src2 = edge_index[0].reshape(NCHUNK, CHR, 128)
    dst2 = edge_index[1].reshape(NCHUNK, CHR, 128)
    typ2 = edge_type.reshape(NCHUNK, CHR, 128)
    zc = jnp.zeros((1024,), jnp.float32)
    zn = jnp.zeros((1024, H), jnp.float32)

    cnt = _cnt_kernel(dst2, typ2, zc)
    inv = _inv(cnt)
    inv2 = jnp.pad(inv, ((0, 0), (0, L - R)))

    basis_m = jnp.transpose(basis1, (0, 2, 1)).reshape(NB * H, N)
    eye = jnp.eye(H, dtype=jnp.float32)
    comp_e = (comp1[:, None, :, None] * eye[None, :, None, :]
              ).reshape(R * H, NB * H)
    w1tab = _w1tab(basis_m, comp_e).reshape(N * R, H)
    xacc = _gss(src2, dst2, typ2, w1tab, inv2, zn)

    w2 = jnp.einsum('rb,bio->rio', comp2, basis2)
    w2m = jnp.transpose(w2, (1, 0, 2)).reshape(H, R * C)
    xw_flat, xr = _xw(xacc, root1, bias1.reshape(1, H), w2m, root2)
    xw = xw_flat.reshape(N * R, C)

    oacc = _gss(src2, dst2, typ2, xw, inv2, zn)
    return _fin(oacc, xr, bias2.reshape(1, C))


# speedup vs baseline: 77.7884x; 1.4662x over previous
import functools

import jax
import jax.numpy as jnp
from jax import lax
from jax.experimental import pallas as pl
from jax.experimental.pallas import tpu as pltpu, tpu_sc as plsc

N = 50000
E = 1600000
R = 8
NB = 30
H = 16
C = 16

NC = 2
NS = 16
L = 16
NW = NC * NS

EROWS = E // 128
CHR = 4
NCHUNK = EROWS // CHR
CH_BASE = NCHUNK // NW
CH_EXTRA = NCHUNK - CH_BASE * NW

ZROW = 3128
CROW_T = (N * R) // NS

_MESH = plsc.VectorSubcoreMesh(core_axis_name="c", subcore_axis_name="s")
_SC_PARAMS = pltpu.CompilerParams(needs_layout_passes=False, use_tc_tiling_on_sc=False)


def _wid():
    return lax.axis_index("s") * NC + lax.axis_index("c")


def _chunk_range(w):
    nch = CH_BASE + (w < CH_EXTRA).astype(jnp.int32)
    start = w * CH_BASE + jnp.minimum(w, CH_EXTRA)
    return start, nch


def _cnt_body(dst2, typ2, zc, out, dstb, typb, kidx, onesb, czbuf,
              sem_e, sem_s, cacc):
    c = lax.axis_index("c")
    s = lax.axis_index("s")
    w = _wid()
    pltpu.sync_copy(zc, czbuf)
    cbase = s * CROW_T
    for k in range(24):
        pltpu.sync_copy(czbuf, cacc.at[pl.ds(cbase + k * 1024, 1024)])
    pltpu.sync_copy(czbuf.at[pl.ds(0, CROW_T - 24 * 1024)],
                    cacc.at[pl.ds(cbase + 24 * 1024, CROW_T - 24 * 1024)])
    for j in range(CHR):
        for g in range(128 // L):
            onesb[j, pl.ds(g * L, L)] = jnp.full((L,), 1.0, jnp.float32)
    plsc.subcore_barrier()

    start, nch = _chunk_range(w)

    def chunk(ci, carry):
        h1 = pltpu.async_copy(dst2.at[start + ci], dstb, sem_e)
        h2 = pltpu.async_copy(typ2.at[start + ci], typb, sem_e)
        h1.wait()
        h2.wait()
        for j in range(CHR):
            for g in range(128 // L):
                sl = pl.ds(g * L, L)
                kidx[j, sl] = dstb[j, sl] * R + typb[j, sl]
        hs = [pltpu.async_copy(onesb.at[j], cacc.at[kidx.at[j]], sem_s,
                               add=True)
              for j in range(CHR)]
        for h in hs:
            h.wait()
        return carry

    lax.fori_loop(0, nch, chunk, 0)
    plsc.subcore_barrier()
    obase = c * (N * R) + s * CROW_T
    for k in range(24):
        pltpu.sync_copy(cacc.at[pl.ds(cbase + k * 1024, 1024)], czbuf)
        pltpu.sync_copy(czbuf, out.at[pl.ds(obase + k * 1024, 1024)])
    tail = CROW_T - 24 * 1024
    pltpu.sync_copy(cacc.at[pl.ds(cbase + 24 * 1024, tail)],
                    czbuf.at[pl.ds(0, tail)])
    pltpu.sync_copy(czbuf.at[pl.ds(0, tail)],
                    out.at[pl.ds(obase + 24 * 1024, tail)])


_cnt_kernel = functools.partial(
    pl.kernel,
    out_type=jax.ShapeDtypeStruct((NC * N * R,), jnp.float32),
    mesh=_MESH,
    compiler_params=_SC_PARAMS,
    scratch_types=[
        pltpu.VMEM((CHR, 128), jnp.int32),
        pltpu.VMEM((CHR, 128), jnp.int32),
        pltpu.VMEM((CHR, 128), jnp.int32),
        pltpu.VMEM((CHR, 128), jnp.float32),
        pltpu.VMEM((1024,), jnp.float32),
        pltpu.SemaphoreType.DMA,
        pltpu.SemaphoreType.DMA,
        pltpu.VMEM_SHARED((N * R,), jnp.float32),
    ],
)(_cnt_body)


def _gss_body(am, bm, src2, dst2, typ2, table, inv2, zn, out,
              srcb, dstb, typb, gidx, rows, normb, normc, zbuf,
              sem_ea, sem_eb, sem_ga, sem_gb, sem_sa, sem_sb, acc):
    c = lax.axis_index("c")
    s = lax.axis_index("s")
    w = _wid()
    pltpu.sync_copy(zn, zbuf)
    zoff = jnp.minimum(s * ZROW, N - ZROW)
    for k in range(3):
        pltpu.sync_copy(zbuf, acc.at[pl.ds(zoff + k * 1024, 1024)])
    pltpu.sync_copy(zbuf.at[pl.ds(0, ZROW - 3 * 1024)],
                    acc.at[pl.ds(zoff + 3 * 1024, ZROW - 3 * 1024)])
    plsc.subcore_barrier()

    start, nch = _chunk_range(w)
    iota = lax.iota(jnp.int32, L)

    def _loads(ci, b, sem):
        return [pltpu.async_copy(src2.at[ci], srcb.at[b], sem),
                pltpu.async_copy(dst2.at[ci], dstb.at[b], sem),
                pltpu.async_copy(typ2.at[ci], typb.at[b], sem)]

    def _fire_gathers(b, sem):
        hs = [pltpu.async_copy(inv2.at[dstb.at[b, j]], normb.at[b, j], sem)
              for j in range(CHR)]
        for j in range(CHR):
            for g in range(128 // L):
                sl = pl.ds(g * L, L)
                gidx[b, j, sl] = typb[b, j, sl] * am + srcb[b, j, sl] * bm
        hs += [pltpu.async_copy(table.at[gidx.at[b, j]], rows.at[b, j], sem)
               for j in range(CHR)]
        return hs

    def _scale(b):
        bv = jnp.full((L,), b, jnp.int32)
        for j in range(CHR):
            jv = jnp.full((L,), j, jnp.int32)
            for g in range(128 // L):
                sl = pl.ds(g * L, L)
                n16 = plsc.load_gather(normb, [bv, jv, iota + (g * L),
                                               typb[b, j, sl]])
                normc[b, j, sl] = n16

            @plsc.parallel_loop(0, 128, unroll=8)
            def _scale_edges(e):
                ev = jnp.full((L,), e, jnp.int32)
                nb = plsc.load_gather(normc, [bv, jv, ev])
                rows[b, j, e] = rows[b, j, e] * nb

    def _fire_scatters(b, sem):
        return [pltpu.async_copy(rows.at[b, j], acc.at[dstb.at[b, j]], sem,
                                 add=True)
                for j in range(CHR)]

    def pair(pi, carry):
        ci_a = start + 2 * pi
        e_a = _loads(ci_a, 0, sem_ea)
        e_b = _loads(ci_a + 1, 1, sem_eb)
        for h in e_a:
            h.wait()
        g_a = _fire_gathers(0, sem_ga)
        for h in e_b:
            h.wait()
        g_b = _fire_gathers(1, sem_gb)
        for h in g_a:
            h.wait()
        _scale(0)
        s_a = _fire_scatters(0, sem_sa)
        for h in g_b:
            h.wait()
        _scale(1)
        s_b = _fire_scatters(1, sem_sb)
        for h in s_a:
            h.wait()
        for h in s_b:
            h.wait()
        return carry

    lax.fori_loop(0, nch // 2, pair, 0)

    @pl.when(nch % 2 == 1)
    def _tail_chunk():
        ci = start + nch - 1
        e0 = _loads(ci, 0, sem_ea)
        for h in e0:
            h.wait()
        g0 = _fire_gathers(0, sem_ga)
        for h in g0:
            h.wait()
        _scale(0)
        s0 = _fire_scatters(0, sem_sa)
        for h in s0:
            h.wait()
    plsc.subcore_barrier()
    for k in range(3):
        pltpu.sync_copy(acc.at[pl.ds(zoff + k * 1024, 1024)], zbuf)
        pltpu.sync_copy(zbuf, out.at[c, pl.ds(zoff + k * 1024, 1024)])
    ztail = ZROW - 3 * 1024
    pltpu.sync_copy(acc.at[pl.ds(zoff + 3 * 1024, ztail)],
                    zbuf.at[pl.ds(0, ztail)])
    pltpu.sync_copy(zbuf.at[pl.ds(0, ztail)],
                    out.at[c, pl.ds(zoff + 3 * 1024, ztail)])


def _make_gss(am, bm):
    return functools.partial(
        pl.kernel,
        out_type=jax.ShapeDtypeStruct((NC, N, H), jnp.float32),
        mesh=_MESH,
        compiler_params=_SC_PARAMS,
        scratch_types=[
            pltpu.VMEM((2, CHR, 128), jnp.int32),
            pltpu.VMEM((2, CHR, 128), jnp.int32),
            pltpu.VMEM((2, CHR, 128), jnp.int32),
            pltpu.VMEM((2, CHR, 128), jnp.int32),
            pltpu.VMEM((2, CHR, 128, L), jnp.float32),
            pltpu.VMEM((2, CHR, 128, L), jnp.float32),
            pltpu.VMEM((2, CHR, 128), jnp.float32),
            pltpu.VMEM((1024, H), jnp.float32),
            pltpu.SemaphoreType.DMA,
            pltpu.SemaphoreType.DMA,
            pltpu.SemaphoreType.DMA,
            pltpu.SemaphoreType.DMA,
            pltpu.SemaphoreType.DMA,
            pltpu.SemaphoreType.DMA,
            pltpu.VMEM_SHARED((N, H), jnp.float32),
        ],
    )(functools.partial(_gss_body, am, bm))


_gss = _make_gss(1, R)


_BN1 = 4096


def _w1tab_body(basis_ref, compe_ref, out_ref):
    out_ref[...] = lax.dot_general(
        basis_ref[...], compe_ref[...],
        dimension_numbers=(((0,), (1,)), ((), ())),
        preferred_element_type=jnp.float32)


def _w1tab(basis_m, comp_e):
    grid = (N + _BN1 - 1) // _BN1
    return pl.pallas_call(
        _w1tab_body,
        grid=(grid,),
        in_specs=[
            pl.BlockSpec((NB * H, _BN1), lambda i: (0, i)),
            pl.BlockSpec((R * H, NB * H), lambda i: (0, 0)),
        ],
        out_specs=pl.BlockSpec((_BN1, R * H), lambda i: (i, 0)),
        out_shape=jax.ShapeDtypeStruct((N, R * H), jnp.float32),
    )(basis_m, comp_e)


_BNI = 2000


def _inv_body(cnt_ref, out_ref):
    c = cnt_ref[0] + cnt_ref[1]
    out_ref[...] = 1.0 / jnp.maximum(c, 1.0)


def _inv(cnt):
    return pl.pallas_call(
        _inv_body,
        grid=(N // _BNI,),
        in_specs=[pl.BlockSpec((NC, _BNI, R), lambda i: (0, i, 0))],
        out_specs=pl.BlockSpec((_BNI, R), lambda i: (i, 0)),
        out_shape=jax.ShapeDtypeStruct((N, R), jnp.float32),
    )(cnt.reshape(NC, N, R))


_BN = 2000


def _xw_body(xacc_ref, root1_ref, b1_ref, w2m_ref, root2_ref, xw_ref, xr_ref):
    x = xacc_ref[0] + xacc_ref[1] + root1_ref[...] + b1_ref[...]
    xw_ref[...] = jnp.dot(x, w2m_ref[...], preferred_element_type=jnp.float32)
    xr_ref[...] = jnp.dot(x, root2_ref[...], preferred_element_type=jnp.float32)


def _xw(xacc, root1, b1, w2m, root2):
    return pl.pallas_call(
        _xw_body,
        grid=(N // _BN,),
        in_specs=[
            pl.BlockSpec((NC, _BN, H), lambda i: (0, i, 0)),
            pl.BlockSpec((_BN, H), lambda i: (i, 0)),
            pl.BlockSpec((1, H), lambda i: (0, 0)),
            pl.BlockSpec((H, R * C), lambda i: (0, 0)),
            pl.BlockSpec((H, C), lambda i: (0, 0)),
        ],
        out_specs=[
            pl.BlockSpec((_BN, R * C), lambda i: (i, 0)),
            pl.BlockSpec((_BN, C), lambda i: (i, 0)),
        ],
        out_shape=[
            jax.ShapeDtypeStruct((N, R * C), jnp.float32),
            jax.ShapeDtypeStruct((N, C), jnp.float32),
        ],
    )(xacc, root1, b1, w2m, root2)


def _fin_body(oacc_ref, xr_ref, b2_ref, out_ref):
    out_ref[...] = oacc_ref[0] + oacc_ref[1] + xr_ref[...] + b2_ref[...]


def _fin(oacc, xr, b2):
    return pl.pallas_call(
        _fin_body,
        grid=(N // _BN,),
        in_specs=[
            pl.BlockSpec((NC, _BN, C), lambda i: (0, i, 0)),
            pl.BlockSpec((_BN, C), lambda i: (i, 0)),
            pl.BlockSpec((1, C), lambda i: (0, 0)),
        ],
        out_specs=pl.BlockSpec((_BN, C), lambda i: (i, 0)),
        out_shape=jax.ShapeDtypeStruct((N, C), jnp.float32),
    )(oacc, xr, b2)


def kernel(edge_index, edge_type, basis1, comp1, root1, bias1,
           basis2, comp2, root2, bias2):
    src2 = edge_index[0].reshape(NCHUNK, CHR, 128)
    dst2 = edge_index[1].reshape(NCHUNK, CHR, 128)
    typ2 = edge_type.reshape(NCHUNK, CHR, 128)
    zc = jnp.zeros((1024,), jnp.float32)
    zn = jnp.zeros((1024, H), jnp.float32)

    cnt = _cnt_kernel(dst2, typ2, zc)
    inv = _inv(cnt)
    inv2 = jnp.pad(inv, ((0, 0), (0, L - R)))

    basis_m = jnp.transpose(basis1, (0, 2, 1)).reshape(NB * H, N)
    eye = jnp.eye(H, dtype=jnp.float32)
    comp_e = (comp1[:, None, :, None] * eye[None, :, None, :]
              ).reshape(R * H, NB * H)
    w1tab = _w1tab(basis_m, comp_e).reshape(N * R, H)
    xacc = _gss(src2, dst2, typ2, w1tab, inv2, zn)

    w2 = jnp.einsum('rb,bio->rio', comp2, basis2)
    w2m = jnp.transpose(w2, (1, 0, 2)).reshape(H, R * C)
    xw_flat, xr = _xw(xacc, root1, bias1.reshape(1, H), w2m, root2)
    xw = xw_flat.reshape(N * R, C)

    oacc = _gss(src2, dst2, typ2, xw, inv2, zn)
    return _fin(oacc, xr, bias2.reshape(1, C))
